# Initial kernel scaffold; baseline (speedup 1.0000x reference)
import jax
import jax.numpy as jnp
from jax.experimental import pallas as pl


def kernel(spatial_x, spatial_global_data, spatial_spatial_edge_index, spatial_spatial_edge_attr, spatial_species_edge_index, spatial_species_edge_attr, species_x, species_x_phylo, species_species_edge_index, species_species_edge_attr, traits_nanmask, sg1_W, sg1_att_src, sg1_att_dst, sg1_We, sg1_att_e, sg1_b, sg2_W, sg2_att_src, sg2_att_dst, sg2_We, sg2_att_e, sg2_b, bip_W, bip_att_src, bip_We, bip_att_e, bip_b, sp1_W, sp1_att_src, sp1_att_dst, sp1_We, sp1_att_e, sp1_b, sp2_W, sp2_att_src, sp2_att_dst, sp2_We, sp2_att_e, sp2_b, fc_W, fc_b):
    raise NotImplementedError("write your pallas kernel here")


# speedup vs baseline: 16.1517x; 16.1517x over previous
import dataclasses
import functools

import jax
import jax.numpy as jnp
from jax import lax
from jax.experimental import pallas as pl
from jax.experimental.pallas import tpu as pltpu
from jax.experimental.pallas import tpu_sc as plsc

F32 = jnp.float32
I32 = jnp.int32

NC, NS, NW, LN = 2, 16, 32, 16
K_EDGE = 1024
RIDX = K_EDGE // 128
BLK = 2000


def _ru(x, m):
    return (x + m - 1) // m * m


def _m8(x):
    return pl.multiple_of(x, 8)


def _sc_params():
    cp = pltpu.CompilerParams()
    if "needs_layout_passes" in pltpu.CompilerParams.__dataclass_fields__:
        cp = dataclasses.replace(cp, needs_layout_passes=False)
    if "use_tc_tiling_on_sc" in pltpu.CompilerParams.__dataclass_fields__:
        cp = dataclasses.replace(cp, use_tc_tiling_on_sc=False)
    return cp


def _sc_mesh():
    return plsc.VectorSubcoreMesh(core_axis_name="c", subcore_axis_name="s")


def _edge_scalar(src2, dst2, ea2, a_src, a_dst, ce16, na, with_loops):
    er = src2.shape[0]
    n_src = a_src.shape[0]
    n_dst = a_dst.shape[0]
    n_chunks = er // (NW * RIDX)
    rows_tile = n_chunks * RIDX
    seg = na // NS

    def body(src_h, dst_h, ea_h, asrc_h, adst_h, ce_h,
             ex_h, s_h, deg_h, bs_h,
             asrc_t, adst_t, ce_t, srcb, dstb, eab, exb, bb, onesb, zb,
             s_acc, deg_acc, bs_acc):
        c = lax.axis_index("c")
        s = lax.axis_index("s")
        wid = s * NC + c

        @pl.loop(0, 128, step=LN)
        def _(i):
            onesb[pl.ds(i, LN)] = jnp.ones((LN,), F32)
            zb[pl.ds(i, LN)] = jnp.zeros((LN,), F32)

        pltpu.sync_copy(asrc_h, asrc_t.at[pl.ds(0, n_src)])
        pltpu.sync_copy(adst_h, adst_t.at[pl.ds(0, n_dst)])
        pltpu.sync_copy(ce_h, ce_t)

        base = s * seg

        @pl.loop(0, seg, step=128)
        def _(i):
            pltpu.sync_copy(zb, s_acc.at[pl.ds(_m8(base + i), 128)])
            if with_loops:
                pltpu.sync_copy(zb, deg_acc.at[pl.ds(_m8(base + i), 128)])
                pltpu.sync_copy(zb, bs_acc.at[pl.ds(_m8(base + i), 128)])

        plsc.subcore_barrier()
        row0 = wid * rows_tile

        @pl.loop(0, n_chunks)
        def _(ch):
            rbase = _m8(row0 + ch * RIDX)
            pltpu.sync_copy(src_h.at[pl.ds(rbase, RIDX)], srcb)
            pltpu.sync_copy(dst_h.at[pl.ds(rbase, RIDX)], dstb)
            pltpu.sync_copy(ea_h.at[pl.ds(rbase, RIDX)], eab)
            cev = ce_t[...]

            @pl.loop(0, RIDX)
            def _(j):
                @pl.loop(0, 128, step=LN)
                def _(q):
                    sv = srcb[j, pl.ds(q, LN)]
                    dv = dstb[j, pl.ds(q, LN)]
                    av = plsc.load_gather(asrc_t, [sv])
                    adv = plsc.load_gather(adst_t, [dv])
                    bv = eab[j, pl.ds(q, LN)] * cev
                    al = av + adv + bv
                    al = jnp.where(al >= 0.0, al, al * 0.2)
                    exb[j, pl.ds(q, LN)] = jnp.exp(al)
                    if with_loops:
                        bb[j, pl.ds(q, LN)] = bv

            pltpu.sync_copy(exb, ex_h.at[pl.ds(rbase, RIDX)])

            @pl.loop(0, RIDX)
            def _(j):
                pltpu.sync_copy(exb.at[j], s_acc.at[dstb.at[j]], add=True)
                if with_loops:
                    pltpu.sync_copy(onesb, deg_acc.at[dstb.at[j]], add=True)
                    pltpu.sync_copy(bb.at[j], bs_acc.at[dstb.at[j]], add=True)

        plsc.subcore_barrier()
        fbase = _m8(c * na + base)
        pltpu.sync_copy(s_acc.at[pl.ds(_m8(base), seg)],
                        s_h.at[pl.ds(fbase, seg)])
        if with_loops:
            pltpu.sync_copy(deg_acc.at[pl.ds(_m8(base), seg)],
                            deg_h.at[pl.ds(fbase, seg)])
            pltpu.sync_copy(bs_acc.at[pl.ds(_m8(base), seg)],
                            bs_h.at[pl.ds(fbase, seg)])

    kern = pl.kernel(
        body,
        out_type=[
            jax.ShapeDtypeStruct((er, 128), F32),
            jax.ShapeDtypeStruct((NC * na,), F32),
            jax.ShapeDtypeStruct((NC * na,), F32),
            jax.ShapeDtypeStruct((NC * na,), F32),
        ],
        mesh=_sc_mesh(),
        scratch_types=[
            pltpu.VMEM((n_src,), F32),
            pltpu.VMEM((n_dst + 128,), F32),
            pltpu.VMEM((LN,), F32),
            pltpu.VMEM((RIDX, 128), I32),
            pltpu.VMEM((RIDX, 128), I32),
            pltpu.VMEM((RIDX, 128), F32),
            pltpu.VMEM((RIDX, 128), F32),
            pltpu.VMEM((RIDX, 128), F32),
            pltpu.VMEM((128,), F32),
            pltpu.VMEM((128,), F32),
            pltpu.VMEM_SHARED((na,), F32),
            pltpu.VMEM_SHARED((na,), F32),
            pltpu.VMEM_SHARED((na,), F32),
        ],
        compiler_params=_sc_params(),
    )
    return kern(src2, dst2, ea2, a_src, a_dst, ce16)


def _edge_agg(src2, dst2, ex2, hc, na):
    er = src2.shape[0]
    n_src = hc.shape[0]
    n_chunks = er // (NW * RIDX)
    rows_tile = n_chunks * RIDX
    seg = na // NS

    def body(src_h, dst_h, ex_h, hc_h, out_h,
             srcb, dstb, exbuf, rows, zb, out_acc):
        c = lax.axis_index("c")
        s = lax.axis_index("s")
        wid = s * NC + c

        @pl.loop(0, 128)
        def _(i):
            zb[i, :] = jnp.zeros((LN,), F32)

        base = s * seg

        @pl.loop(0, seg, step=128)
        def _(i):
            pltpu.sync_copy(zb, out_acc.at[pl.ds(_m8(base + i), 128)])

        plsc.subcore_barrier()
        row0 = wid * rows_tile

        @pl.loop(0, n_chunks)
        def _(ch):
            rbase = _m8(row0 + ch * RIDX)
            pltpu.sync_copy(src_h.at[pl.ds(rbase, RIDX)], srcb)
            pltpu.sync_copy(dst_h.at[pl.ds(rbase, RIDX)], dstb)
            pltpu.sync_copy(ex_h.at[pl.ds(rbase, RIDX)], exbuf)

            @pl.loop(0, RIDX)
            def _(j):
                pltpu.sync_copy(hc_h.at[srcb.at[j]], rows)

                @pl.loop(0, 128, step=LN)
                def _(q):
                    ev = exbuf[j, pl.ds(q, LN)]
                    for k in range(LN):
                        rows[q + k, :] = rows[q + k, :] * ev[k]

                pltpu.sync_copy(rows, out_acc.at[dstb.at[j]], add=True)

        plsc.subcore_barrier()
        pltpu.sync_copy(out_acc.at[pl.ds(_m8(base), seg)],
                        out_h.at[pl.ds(_m8(c * na + base), seg)])

    kern = pl.kernel(
        body,
        out_type=jax.ShapeDtypeStruct((NC * na, LN), F32),
        mesh=_sc_mesh(),
        scratch_types=[
            pltpu.VMEM((RIDX, 128), I32),
            pltpu.VMEM((RIDX, 128), I32),
            pltpu.VMEM((RIDX, 128), F32),
            pltpu.VMEM((128, LN), F32),
            pltpu.VMEM((128, LN), F32),
            pltpu.VMEM_SHARED((na, LN), F32),
        ],
        compiler_params=_sc_params(),
    )
    return kern(src2, dst2, ex2, hc)


def _pre(x, W, att_s, att_d):
    n, din = x.shape

    def body(x_r, w_r, as_r, ad_r, a_s, a_d, h0, h1, h2, h3):
        h = jnp.dot(x_r[...], w_r[...], preferred_element_type=F32)
        a_s[...] = jnp.dot(h, as_r[...], preferred_element_type=F32)
        a_d[...] = jnp.dot(h, ad_r[...], preferred_element_type=F32)
        for k, hr in enumerate((h0, h1, h2, h3)):
            hr[...] = h[:, k * LN:(k + 1) * LN]

    outs = pl.pallas_call(
        body,
        grid=(n // BLK,),
        in_specs=[
            pl.BlockSpec((BLK, din), lambda i: (i, 0)),
            pl.BlockSpec((din, 64), lambda i: (0, 0)),
            pl.BlockSpec((64, 1), lambda i: (0, 0)),
            pl.BlockSpec((64, 1), lambda i: (0, 0)),
        ],
        out_specs=[pl.BlockSpec((BLK, 1), lambda i: (i, 0))] * 2
        + [pl.BlockSpec((BLK, LN), lambda i: (i, 0))] * 4,
        out_shape=[jax.ShapeDtypeStruct((n, 1), F32)] * 2
        + [jax.ShapeDtypeStruct((n, LN), F32)] * 4,
    )(x, W, att_s.reshape(64, 1), att_d.reshape(64, 1))
    return outs[0], outs[1], outs[2:]


def _glue(s2, deg2, bs2, a_s, a_d, hcs, ocs, bias, n, loops, relu, final):
    na = s2.shape[1]

    def body(*refs):
        it = iter(refs)
        s2_r = next(it)[...][:, :, 0]
        deg_r = next(it)[...][:, :, 0]
        bs_r = next(it)[...][:, :, 0]
        as_r = next(it)[...]
        ad_r = next(it)[...]
        h_r = [next(it)[...] for _ in range(4)]
        o_r = [next(it)[...] for _ in range(4)]
        b_r = next(it)[...]
        if final is not None:
            fw_r = next(it)[...]
            fb_r = next(it)[...]
            m_r = next(it)[...]
        out_r = next(it)

        s = s2_r[0] + s2_r[1]
        if loops:
            deg = jnp.maximum(deg_r[0] + deg_r[1], 1.0)
            bavg = (bs_r[0] + bs_r[1]) / deg
            al = as_r[:, 0] + ad_r[:, 0] + bavg
            al = jnp.where(al >= 0.0, al, al * 0.2)
            exl = jnp.exp(al)
            stot = s + exl + 1e-16
            num = [o_r[k][0] + o_r[k][1] + exl[:, None] * h_r[k]
                   for k in range(4)]
        else:
            stot = s + 1e-16
            num = [o_r[k][0] + o_r[k][1] for k in range(4)]
        y = jnp.concatenate(num, axis=1) / stot[:, None] + b_r
        if relu:
            y = jnp.maximum(y, 0.0)
        if final is not None:
            y = jnp.dot(y, fw_r, preferred_element_type=F32) + fb_r
            y = y * (1.0 - m_r.astype(F32))
        out_r[...] = y

    in_specs = [
        pl.BlockSpec((2, BLK, 1), lambda i: (0, i, 0)),
        pl.BlockSpec((2, BLK, 1), lambda i: (0, i, 0)),
        pl.BlockSpec((2, BLK, 1), lambda i: (0, i, 0)),
        pl.BlockSpec((BLK, 1), lambda i: (i, 0)),
        pl.BlockSpec((BLK, 1), lambda i: (i, 0)),
    ]
    in_specs += [pl.BlockSpec((BLK, LN), lambda i: (i, 0))] * 4
    in_specs += [pl.BlockSpec((2, BLK, LN), lambda i: (0, i, 0))] * 4
    in_specs += [pl.BlockSpec((1, 64), lambda i: (0, 0))]
    args = [s2.reshape(2, na, 1), deg2.reshape(2, na, 1),
            bs2.reshape(2, na, 1), a_s, a_d, *hcs, *ocs, bias.reshape(1, 64)]
    if final is not None:
        fw, fb, msk = final
        in_specs += [
            pl.BlockSpec((64, 32), lambda i: (0, 0)),
            pl.BlockSpec((1, 32), lambda i: (0, 0)),
            pl.BlockSpec((BLK, 32), lambda i: (i, 0)),
        ]
        args += [fw, fb.reshape(1, 32), msk]
        out_shape = jax.ShapeDtypeStruct((n, 32), F32)
        out_spec = pl.BlockSpec((BLK, 32), lambda i: (i, 0))
    else:
        out_shape = jax.ShapeDtypeStruct((n, 64), F32)
        out_spec = pl.BlockSpec((BLK, 64), lambda i: (i, 0))

    return pl.pallas_call(
        body,
        grid=(n // BLK,),
        in_specs=in_specs,
        out_specs=out_spec,
        out_shape=out_shape,
    )(*args)


def _prep_edges(src, dst, ea, n_dst, e_pad):
    pad = e_pad - src.shape[0]
    src_p = jnp.concatenate([src, jnp.zeros((pad,), I32)]).reshape(-1, 128)
    dst_p = jnp.concatenate([dst, jnp.full((pad,), n_dst, I32)]).reshape(-1, 128)
    ea_p = jnp.concatenate([ea.reshape(-1), jnp.zeros((pad,), F32)]
                           ).reshape(-1, 128)
    return src_p, dst_p, ea_p


def _gat_layer(x, W, att_s, att_d, We, att_e, bias, edges, n_dst, na,
               loops, relu, a_dst_override=None, glue_nd_dummy=None,
               final=None):
    src2, dst2, ea2 = edges
    a_s, a_d, hcs = _pre(x, W, att_s, att_d)
    ce = jnp.dot(We.reshape(-1), att_e)
    ce16 = jnp.full((LN,), ce, F32)
    a_src_flat = a_s.reshape(-1)
    a_dst_flat = a_d.reshape(-1) if a_dst_override is None else a_dst_override
    ex2, s2, deg2, bs2 = _edge_scalar(src2, dst2, ea2, a_src_flat, a_dst_flat,
                                      ce16, na, loops)
    s2, deg2, bs2 = (v.reshape(NC, na) for v in (s2, deg2, bs2))
    ocs = [_edge_agg(src2, dst2, ex2, hcs[k], na).reshape(NC, na, LN)
           for k in range(4)]
    if glue_nd_dummy is not None:
        a_s, a_d, hcs = glue_nd_dummy
    return _glue(s2, deg2, bs2, a_s, a_d, hcs, ocs, bias, n_dst, loops, relu,
                 final)


def kernel(spatial_x, spatial_global_data, spatial_spatial_edge_index,
           spatial_spatial_edge_attr, spatial_species_edge_index,
           spatial_species_edge_attr, species_x, species_x_phylo,
           species_species_edge_index, species_species_edge_attr,
           traits_nanmask, sg1_W, sg1_att_src, sg1_att_dst, sg1_We, sg1_att_e,
           sg1_b, sg2_W, sg2_att_src, sg2_att_dst, sg2_We, sg2_att_e, sg2_b,
           bip_W, bip_att_src, bip_We, bip_att_e, bip_b, sp1_W, sp1_att_src,
           sp1_att_dst, sp1_We, sp1_att_e, sp1_b, sp2_W, sp2_att_src,
           sp2_att_dst, sp2_We, sp2_att_e, sp2_b, fc_W, fc_b):
    n_sp = spatial_x.shape[0]
    n_s = species_x.shape[0]
    na_sp = _ru(n_sp + 1, NS * 128)
    na_s = _ru(n_s + 1, NS * 128)

    e_sp = spatial_spatial_edge_index.shape[1]
    e_bip = spatial_species_edge_index.shape[1]
    e_s = species_species_edge_index.shape[1]
    pad_unit = NW * K_EDGE
    sp_edges = _prep_edges(spatial_spatial_edge_index[0],
                           spatial_spatial_edge_index[1],
                           spatial_spatial_edge_attr, n_sp,
                           _ru(e_sp, pad_unit))
    bip_edges = _prep_edges(spatial_species_edge_index[0],
                            spatial_species_edge_index[1],
                            spatial_species_edge_attr, n_s,
                            _ru(e_bip, pad_unit))
    s_edges = _prep_edges(species_species_edge_index[0],
                          species_species_edge_index[1],
                          species_species_edge_attr, n_s,
                          _ru(e_s, pad_unit))

    xi = jnp.concatenate([spatial_x, spatial_global_data], axis=1)
    h = _gat_layer(xi, sg1_W, sg1_att_src, sg1_att_dst, sg1_We, sg1_att_e,
                   sg1_b, sp_edges, n_sp, na_sp, loops=True, relu=True)
    space_emb = _gat_layer(h, sg2_W, sg2_att_src, sg2_att_dst, sg2_We,
                           sg2_att_e, sg2_b, sp_edges, n_sp, na_sp,
                           loops=True, relu=True)
    zeros_nd = jnp.zeros((n_s,), F32)
    dummy = (jnp.zeros((n_s, 1), F32), jnp.zeros((n_s, 1), F32),
             [jnp.zeros((n_s, LN), F32)] * 4)
    s2s = _gat_layer(space_emb, bip_W, bip_att_src, jnp.zeros((64,), F32),
                     bip_We, bip_att_e, bip_b, bip_edges, n_s, na_s,
                     loops=False, relu=True, a_dst_override=zeros_nd,
                     glue_nd_dummy=dummy)
    si = jnp.concatenate([s2s, species_x, species_x_phylo], axis=1)
    g = _gat_layer(si, sp1_W, sp1_att_src, sp1_att_dst, sp1_We, sp1_att_e,
                   sp1_b, s_edges, n_s, na_s, loops=True, relu=True)
    out = _gat_layer(g, sp2_W, sp2_att_src, sp2_att_dst, sp2_We, sp2_att_e,
                     sp2_b, s_edges, n_s, na_s, loops=True, relu=True,
                     final=(fc_W, fc_b, traits_nanmask))
    return out
